# baseline (speedup 1.0000x reference)
import functools

import jax
import jax.numpy as jnp
import numpy as np
from jax import lax
from jax.experimental import pallas as pl
from jax.experimental.pallas import tpu as pltpu
from jax.experimental.pallas import tpu_sc as plsc

VOCAB = 1000000
D = 64
B = 16384
H = 200

_NC = 2
_NS = 16
_NW = _NC * _NS
_SPW = B // _NW
_S = 2
_CHUNK = 40
_NPS = H // _CHUNK
_NCH = _S * _NPS
_GROUPS = _SPW // _S
_NBUF = 2
_NIB = 4

_PERM = np.concatenate(
    [np.arange(0, 32, 2), np.arange(1, 32, 2),
     np.arange(32, 64, 2), np.arange(33, 64, 2)]
)


def _pool_body(x_hbm, emb_hbm, out_hbm, idx_v, rows_v, out_v,
               sr0, sr1, si0, si1, si2, si3):
    wid = lax.axis_index("s") * _NC + lax.axis_index("c")
    base = wid * _SPW
    sems_r = (sr0, sr1)
    sems_i = (si0, si1, si2, si3)

    def idx_fetch(g, ib):
        for s in range(_S):
            for c in range(_NPS):
                pltpu.async_copy(
                    x_hbm.at[base + g * _S + s, pl.ds(c * _CHUNK, _CHUNK)],
                    idx_v.at[ib, s * _NPS + c],
                    sems_i[ib],
                )

    def wait_idx(ib):
        pltpu.make_async_copy(
            x_hbm.at[pl.ds(0, _NCH), pl.ds(0, _CHUNK)], idx_v.at[ib],
            sems_i[ib],
        ).wait()

    def fire(ib, rb):
        for j in range(_NCH):
            pltpu.async_copy(
                emb_hbm.at[idx_v.at[ib, j]],
                rows_v.at[rb, pl.ds(j * _CHUNK, _CHUNK)],
                sems_r[rb],
            )

    def wait_rows(rb):
        pltpu.make_async_copy(
            emb_hbm.at[pl.ds(0, _S * H)], rows_v.at[rb], sems_r[rb]
        ).wait()

    def reduce(g, rb):
        for s in range(_S):
            r0 = s * H

            def red(r, acc):
                w0 = rows_v[rb, r0 + r, pl.ds(0, 32)]
                w1 = rows_v[rb, r0 + r, pl.ds(32, 32)]
                a0, b0 = plsc.unpack(w0, format=plsc.PackFormat.INTERLEAVED)
                a1, b1 = plsc.unpack(w1, format=plsc.PackFormat.INTERLEAVED)
                return (acc[0] + a0, acc[1] + b0, acc[2] + a1, acc[3] + b1)

            acc = lax.fori_loop(
                0, H, red,
                tuple(jnp.zeros((16,), jnp.float32) for _ in range(4)),
                unroll=8,
            )
            for k in range(4):
                out_v[g * _S + s, pl.ds(16 * k, 16)] = acc[k]

    for s in range(_S):
        for c in range(_NPS):
            pltpu.sync_copy(
                x_hbm.at[base + s, pl.ds(c * _CHUNK, _CHUNK)],
                idx_v.at[0, s * _NPS + c],
            )
    fire(0, 0)
    for p in range(1, _NIB):
        idx_fetch(p, p)

    def quad(qq, carry):
        for b in range(_NIB):
            g = _NIB * qq + b
            rb = b % _NBUF
            nrb = (b + 1) % _NBUF

            @pl.when(g + 1 < _GROUPS)
            def _():
                wait_idx((b + 1) % _NIB)
                fire((b + 1) % _NIB, nrb)

            wait_rows(rb)

            @pl.when(g + _NIB < _GROUPS)
            def _():
                idx_fetch(g + _NIB, b)

            reduce(g, rb)
        return carry

    lax.fori_loop(0, _GROUPS // _NIB, quad, 0)
    pltpu.sync_copy(out_v, out_hbm.at[pl.ds(base, _SPW)])


def _pool(x, emb):
    mesh = plsc.VectorSubcoreMesh(core_axis_name="c", subcore_axis_name="s")
    fn = functools.partial(
        pl.kernel,
        mesh=mesh,
        compiler_params=pltpu.CompilerParams(
            use_tc_tiling_on_sc=False, needs_layout_passes=False
        ),
        out_type=jax.ShapeDtypeStruct((B, D), jnp.float32),
        scratch_types=[
            pltpu.VMEM((_NIB, _NCH, _CHUNK), jnp.int32),
            pltpu.VMEM((_NBUF, _S * H, D), jnp.bfloat16),
            pltpu.VMEM((_SPW, D), jnp.float32),
        ] + [pltpu.SemaphoreType.DMA] * (_NBUF + _NIB),
    )(_pool_body)
    return fn(x, emb)


def _dense_body(h_ref, w_ref, b_ref, o_ref):
    o_ref[...] = jnp.tanh(
        jnp.dot(h_ref[...], w_ref[...], preferred_element_type=jnp.float32)
        + b_ref[...]
    )


def _dense(h, W, b):
    blk = 2048
    return pl.pallas_call(
        _dense_body,
        grid=(B // blk,),
        in_specs=[
            pl.BlockSpec((blk, D), lambda i: (i, 0)),
            pl.BlockSpec((D, D), lambda i: (0, 0)),
            pl.BlockSpec((1, D), lambda i: (0, 0)),
        ],
        out_specs=pl.BlockSpec((blk, D), lambda i: (i, 0)),
        out_shape=jax.ShapeDtypeStruct((B, D), jnp.float32),
    )(h, W, b.reshape(1, D))


def kernel(x, emb, W, b):
    emb16 = emb.reshape(VOCAB * D).astype(jnp.bfloat16).reshape(VOCAB, D)
    pooled = _pool(x, emb16)
    return _dense(pooled, W[jnp.asarray(_PERM)], b)


# speedup vs baseline: 1.1052x; 1.1052x over previous
import functools

import jax
import jax.numpy as jnp
from jax import lax
from jax.experimental import pallas as pl
from jax.experimental.pallas import tpu as pltpu
from jax.experimental.pallas import tpu_sc as plsc

VOCAB = 1000000
D = 64
B = 16384
H = 200

_NC = 2
_NS = 16
_NW = _NC * _NS
_SPW = B // _NW
_S = 2
_CHUNK = 40
_NPS = H // _CHUNK
_NCH = _S * _NPS
_GROUPS = _SPW // _S
_NBUF = 2
_NIB = 4


def _pool_body(x_hbm, emb_hbm, out_hbm, idx_v, rows_v, out_v,
               sr0, sr1, si0, si1, si2, si3):
    wid = lax.axis_index("s") * _NC + lax.axis_index("c")
    base = wid * _SPW
    sems_r = (sr0, sr1)
    sems_i = (si0, si1, si2, si3)

    def idx_fetch(g, ib):
        for s in range(_S):
            for c in range(_NPS):
                pltpu.async_copy(
                    x_hbm.at[base + g * _S + s, pl.ds(c * _CHUNK, _CHUNK)],
                    idx_v.at[ib, s * _NPS + c],
                    sems_i[ib],
                )

    def wait_idx(ib):
        pltpu.make_async_copy(
            x_hbm.at[pl.ds(0, _NCH), pl.ds(0, _CHUNK)], idx_v.at[ib],
            sems_i[ib],
        ).wait()

    def fire(ib, rb):
        for j in range(_NCH):
            pltpu.async_copy(
                emb_hbm.at[idx_v.at[ib, j]],
                rows_v.at[rb, pl.ds(j * _CHUNK, _CHUNK)],
                sems_r[rb],
            )

    def wait_rows(rb):
        pltpu.make_async_copy(
            emb_hbm.at[pl.ds(0, _S * H)], rows_v.at[rb], sems_r[rb]
        ).wait()

    def reduce(g, rb):
        for s in range(_S):
            r0 = s * H

            def red(r, acc):
                return tuple(
                    acc[k] + rows_v[rb, r0 + r, pl.ds(16 * k, 16)]
                    for k in range(4)
                )

            acc = lax.fori_loop(
                0, H, red,
                tuple(jnp.zeros((16,), jnp.float32) for _ in range(4)),
                unroll=8,
            )
            for k in range(4):
                out_v[g * _S + s, pl.ds(16 * k, 16)] = acc[k]

    for s in range(_S):
        for c in range(_NPS):
            pltpu.sync_copy(
                x_hbm.at[base + s, pl.ds(c * _CHUNK, _CHUNK)],
                idx_v.at[0, s * _NPS + c],
            )
    fire(0, 0)
    for p in range(1, _NIB):
        idx_fetch(p, p)

    def quad(qq, carry):
        for b in range(_NIB):
            g = _NIB * qq + b
            rb = b % _NBUF
            nrb = (b + 1) % _NBUF

            @pl.when(g + 1 < _GROUPS)
            def _():
                wait_idx((b + 1) % _NIB)
                fire((b + 1) % _NIB, nrb)

            wait_rows(rb)

            @pl.when(g + _NIB < _GROUPS)
            def _():
                idx_fetch(g + _NIB, b)

            reduce(g, rb)
        return carry

    lax.fori_loop(0, _GROUPS // _NIB, quad, 0)
    pltpu.sync_copy(out_v, out_hbm.at[pl.ds(base, _SPW)])


def _pool(x, emb):
    mesh = plsc.VectorSubcoreMesh(core_axis_name="c", subcore_axis_name="s")
    fn = functools.partial(
        pl.kernel,
        mesh=mesh,
        compiler_params=pltpu.CompilerParams(use_tc_tiling_on_sc=False),
        out_type=jax.ShapeDtypeStruct((B, D), jnp.float32),
        scratch_types=[
            pltpu.VMEM((_NIB, _NCH, _CHUNK), jnp.int32),
            pltpu.VMEM((_NBUF, _S * H, D), jnp.float32),
            pltpu.VMEM((_SPW, D), jnp.float32),
        ] + [pltpu.SemaphoreType.DMA] * (_NBUF + _NIB),
    )(_pool_body)
    return fn(x, emb)


def _dense_body(h_ref, w_ref, b_ref, o_ref):
    o_ref[...] = jnp.tanh(
        jnp.dot(h_ref[...], w_ref[...], preferred_element_type=jnp.float32)
        + b_ref[...]
    )


def _dense(h, W, b):
    blk = 2048
    return pl.pallas_call(
        _dense_body,
        grid=(B // blk,),
        in_specs=[
            pl.BlockSpec((blk, D), lambda i: (i, 0)),
            pl.BlockSpec((D, D), lambda i: (0, 0)),
            pl.BlockSpec((1, D), lambda i: (0, 0)),
        ],
        out_specs=pl.BlockSpec((blk, D), lambda i: (i, 0)),
        out_shape=jax.ShapeDtypeStruct((B, D), jnp.float32),
    )(h, W, b.reshape(1, D))


def kernel(x, emb, W, b):
    pooled = _pool(x, emb)
    return _dense(pooled, W, b)


# speedup vs baseline: 1.1130x; 1.0071x over previous
import functools

import jax
import jax.numpy as jnp
from jax import lax
from jax.experimental import pallas as pl
from jax.experimental.pallas import tpu as pltpu
from jax.experimental.pallas import tpu_sc as plsc

VOCAB = 1000000
D = 64
B = 16384
H = 200

_NC = 2
_NS = 16
_NW = _NC * _NS
_SPW = B // _NW
_S = 2
_C0, _C1 = 128, 72
_GROUPS = _SPW // _S
_NBUF = 2
_NIB = 4


def _pool_body(x_hbm, emb_hbm, out_hbm, idxa_v, idxb_v, rows_v, out_v,
               sr0, sr1, si0, si1, si2, si3):
    wid = lax.axis_index("s") * _NC + lax.axis_index("c")
    base = wid * _SPW
    sems_r = (sr0, sr1)
    sems_i = (si0, si1, si2, si3)

    def idx_fetch(g, ib):
        for s in range(_S):
            m = base + g * _S + s
            pltpu.async_copy(
                x_hbm.at[m, pl.ds(0, _C0)], idxa_v.at[ib, s], sems_i[ib]
            )
            pltpu.async_copy(
                x_hbm.at[m, pl.ds(_C0, _C1)], idxb_v.at[ib, s], sems_i[ib]
            )

    def wait_idx(ib):
        pltpu.make_async_copy(
            x_hbm.at[pl.ds(0, _S), pl.ds(0, _C0)], idxa_v.at[ib], sems_i[ib]
        ).wait()
        pltpu.make_async_copy(
            x_hbm.at[pl.ds(0, _S), pl.ds(0, _C1)], idxb_v.at[ib], sems_i[ib]
        ).wait()

    def fire(ib, rb):
        for s in range(_S):
            pltpu.async_copy(
                emb_hbm.at[idxa_v.at[ib, s]],
                rows_v.at[rb, pl.ds(s * H, _C0)],
                sems_r[rb],
            )
            pltpu.async_copy(
                emb_hbm.at[idxb_v.at[ib, s]],
                rows_v.at[rb, pl.ds(s * H + _C0, _C1)],
                sems_r[rb],
            )

    def wait_rows(rb):
        pltpu.make_async_copy(
            emb_hbm.at[pl.ds(0, _S * H)], rows_v.at[rb], sems_r[rb]
        ).wait()

    def reduce(g, rb):
        for s in range(_S):
            r0 = s * H

            def red(r, acc):
                return tuple(
                    acc[k] + rows_v[rb, r0 + r, pl.ds(16 * k, 16)]
                    for k in range(4)
                )

            acc = lax.fori_loop(
                0, H, red,
                tuple(jnp.zeros((16,), jnp.float32) for _ in range(4)),
                unroll=8,
            )
            for k in range(4):
                out_v[g * _S + s, pl.ds(16 * k, 16)] = acc[k]

    for s in range(_S):
        pltpu.sync_copy(x_hbm.at[base + s, pl.ds(0, _C0)], idxa_v.at[0, s])
        pltpu.sync_copy(x_hbm.at[base + s, pl.ds(_C0, _C1)], idxb_v.at[0, s])
    fire(0, 0)
    for p in range(1, _NIB):
        idx_fetch(p, p)

    def quad(qq, carry):
        for b in range(_NIB):
            g = _NIB * qq + b
            rb = b % _NBUF
            nrb = (b + 1) % _NBUF

            @pl.when(g + 1 < _GROUPS)
            def _():
                wait_idx((b + 1) % _NIB)
                fire((b + 1) % _NIB, nrb)

            wait_rows(rb)

            @pl.when(g + _NIB < _GROUPS)
            def _():
                idx_fetch(g + _NIB, b)

            reduce(g, rb)
        return carry

    lax.fori_loop(0, _GROUPS // _NIB, quad, 0)
    pltpu.sync_copy(out_v, out_hbm.at[pl.ds(base, _SPW)])


def _pool(x, emb):
    mesh = plsc.VectorSubcoreMesh(core_axis_name="c", subcore_axis_name="s")
    fn = functools.partial(
        pl.kernel,
        mesh=mesh,
        compiler_params=pltpu.CompilerParams(use_tc_tiling_on_sc=False),
        out_type=jax.ShapeDtypeStruct((B, D), jnp.float32),
        scratch_types=[
            pltpu.VMEM((_NIB, _S, _C0), jnp.int32),
            pltpu.VMEM((_NIB, _S, _C1), jnp.int32),
            pltpu.VMEM((_NBUF, _S * H, D), jnp.float32),
            pltpu.VMEM((_SPW, D), jnp.float32),
        ] + [pltpu.SemaphoreType.DMA] * (_NBUF + _NIB),
    )(_pool_body)
    return fn(x, emb)


def _dense_body(h_ref, w_ref, b_ref, o_ref):
    o_ref[...] = jnp.tanh(
        jnp.dot(h_ref[...], w_ref[...], preferred_element_type=jnp.float32)
        + b_ref[...]
    )


def _dense(h, W, b):
    blk = 2048
    return pl.pallas_call(
        _dense_body,
        grid=(B // blk,),
        in_specs=[
            pl.BlockSpec((blk, D), lambda i: (i, 0)),
            pl.BlockSpec((D, D), lambda i: (0, 0)),
            pl.BlockSpec((1, D), lambda i: (0, 0)),
        ],
        out_specs=pl.BlockSpec((blk, D), lambda i: (i, 0)),
        out_shape=jax.ShapeDtypeStruct((B, D), jnp.float32),
    )(h, W, b.reshape(1, D))


def kernel(x, emb, W, b):
    pooled = _pool(x, emb)
    return _dense(pooled, W, b)
